# baseline (speedup 1.0000x reference)
import functools

import jax
import jax.numpy as jnp
from jax import lax
from jax.experimental import pallas as pl
from jax.experimental.pallas import tpu as pltpu
from jax.experimental.pallas import tpu_sc as plsc

V = 1000000
D = 64
LABELS = 2
B = 4096
L = 200

NW = 32
BPW = B // NW
LA = 96
LB = L - LA

VBLK = 8192

_mesh = plsc.VectorSubcoreMesh(core_axis_name="c", subcore_axis_name="s")


def _tw_body(w_ref, t_ref, o0_ref, o1_ref):
    t = t_ref[...]
    w0 = w_ref[0, :]
    w1 = w_ref[1, :]
    o0_ref[...] = jnp.sum(t * w0[None, :], axis=1)
    o1_ref[...] = jnp.sum(t * w1[None, :], axis=1)


_tw_kernel = pl.pallas_call(
    _tw_body,
    grid=((V + VBLK - 1) // VBLK,),
    in_specs=[
        pl.BlockSpec((8, D), lambda i: (0, 0)),
        pl.BlockSpec((VBLK, D), lambda i: (i, 0)),
    ],
    out_specs=[
        pl.BlockSpec((VBLK,), lambda i: (i,)),
        pl.BlockSpec((VBLK,), lambda i: (i,)),
    ],
    out_shape=[
        jax.ShapeDtypeStruct((V,), jnp.float32),
        jax.ShapeDtypeStruct((V,), jnp.float32),
    ],
)


@functools.partial(
    pl.kernel,
    out_type=jax.ShapeDtypeStruct((B * LABELS,), jnp.float32),
    mesh=_mesh,
    compiler_params=pltpu.CompilerParams(
        needs_layout_passes=False, use_tc_tiling_on_sc=False),
    scratch_types=[
        pltpu.VMEM((2, L), jnp.int32),
        pltpu.VMEM((2, L), jnp.float32),
        pltpu.VMEM((2, L), jnp.float32),
        pltpu.VMEM((BPW + 16,), jnp.float32),
        pltpu.VMEM((16,), jnp.float32),
        pltpu.VMEM((BPW * 16,), jnp.float32),
        pltpu.VMEM((BPW * LABELS,), jnp.float32),
        pltpu.SemaphoreType.DMA,
        pltpu.SemaphoreType.DMA,
        pltpu.SemaphoreType.DMA,
        pltpu.SemaphoreType.DMA,
    ],
)
def _embd_sc_kernel(x_hbm, xlen_hbm, tw0_hbm, tw1_hbm, b_hbm, out_hbm,
                    idx_v, g0, g1, xlen_v, b_v, tmp_v, out_v,
                    gsem0, gsem1, isem0, isem1):
    wid = lax.axis_index("s") * 2 + lax.axis_index("c")
    base = wid * BPW

    pltpu.sync_copy(xlen_hbm.at[pl.ds(base, BPW)], xlen_v.at[pl.ds(0, BPW)])
    pltpu.sync_copy(b_hbm, b_v)

    gsem = (gsem0, gsem1)
    isem = (isem0, isem1)

    bvec = b_v[...]
    b0 = bvec[0]
    b1 = bvec[1]
    lane = lax.iota(jnp.int32, 16)
    tail_mask = lane >= 8

    def idx_copy(r, slot):
        r = jnp.minimum(r, BPW - 1)
        return pltpu.make_async_copy(
            x_hbm.at[pl.ds((base + r) * L, L)], idx_v.at[slot], isem[slot])

    def gathers(slot):
        yield pltpu.make_async_copy(
            tw0_hbm.at[idx_v.at[slot, pl.ds(0, LA)]],
            g0.at[slot, pl.ds(0, LA)], gsem[slot])
        yield pltpu.make_async_copy(
            tw0_hbm.at[idx_v.at[slot, pl.ds(LA, LB)]],
            g0.at[slot, pl.ds(LA, LB)], gsem[slot])
        yield pltpu.make_async_copy(
            tw1_hbm.at[idx_v.at[slot, pl.ds(0, LA)]],
            g1.at[slot, pl.ds(0, LA)], gsem[slot])
        yield pltpu.make_async_copy(
            tw1_hbm.at[idx_v.at[slot, pl.ds(LA, LB)]],
            g1.at[slot, pl.ds(LA, LB)], gsem[slot])

    def gather_start(slot):
        for cp in gathers(slot):
            cp.start()

    def gather_wait(slot):
        for cp in gathers(slot):
            cp.wait()

    def reduce_row(slot, r):
        z = jnp.zeros((16,), jnp.float32)

        def red(i, accs):
            a0, a1 = accs
            a0 = a0 + g0[slot, pl.ds(i * 16, 16)]
            a1 = a1 + g1[slot, pl.ds(i * 16, 16)]
            return (a0, a1)

        a0, a1 = lax.fori_loop(0, L // 16, red, (z, z), unroll=12)
        t0 = g0[slot, pl.ds(L - 16, 16)]
        t1 = g1[slot, pl.ds(L - 16, 16)]
        a0 = a0 + jnp.where(tail_mask, t0, 0.0)
        a1 = a1 + jnp.where(tail_mask, t1, 0.0)

        inv = (1.0 / xlen_v[pl.ds(r, 16)])[0]
        s0 = jnp.sum(a0) * inv + b0
        s1 = jnp.sum(a1) * inv + b1
        vres = jnp.where(lane == 0, jnp.full((16,), s0, jnp.float32),
                         jnp.full((16,), s1, jnp.float32))
        vres = 1.0 / (1.0 + jnp.exp(-vres))
        tmp_v[pl.ds(r * 16, 16)] = vres

    idx_copy(0, 0).start()
    idx_copy(1, 1).start()
    idx_copy(0, 0).wait()
    gather_start(0)
    idx_copy(1, 1).wait()
    gather_start(1)

    def pair_body(rp, _):
        r0 = 2 * rp
        gather_wait(0)
        idx_copy(r0 + 2, 0).start()
        reduce_row(0, r0)
        gather_wait(1)
        idx_copy(r0 + 3, 1).start()
        idx_copy(r0 + 2, 0).wait()
        gather_start(0)
        reduce_row(1, r0 + 1)
        idx_copy(r0 + 3, 1).wait()
        gather_start(1)
        return 0

    lax.fori_loop(0, BPW // 2, pair_body, 0)

    gather_wait(0)
    gather_wait(1)

    def pack_body(g, _):
        idx = ((lane >> 1) + g * 8) * 16 + (lane & 1)
        out_v[pl.ds(g * 16, 16)] = plsc.load_gather(tmp_v, [idx])
        return 0

    lax.fori_loop(0, (BPW * LABELS) // 16, pack_body, 0)

    pltpu.sync_copy(out_v, out_hbm.at[pl.ds(base * LABELS, BPW * LABELS)])


def kernel(x, xlength, embd_table, fc_W, fc_b):
    w_pad = jnp.zeros((8, D), jnp.float32).at[:LABELS].set(fc_W)
    tw0, tw1 = _tw_kernel(w_pad, embd_table)
    x_flat = x.reshape(B * L)
    xlen_flat = xlength.reshape(B)
    b_pad = jnp.zeros((16,), jnp.float32).at[:LABELS].set(fc_b)
    out_flat = _embd_sc_kernel(x_flat, xlen_flat, tw0, tw1, b_pad)
    return out_flat.reshape(B, LABELS)


# speedup vs baseline: 1.6419x; 1.6419x over previous
import functools

import jax
import jax.numpy as jnp
from jax import lax
from jax.experimental import pallas as pl
from jax.experimental.pallas import tpu as pltpu
from jax.experimental.pallas import tpu_sc as plsc

V = 1000000
D = 64
LABELS = 2
B = 4096
L = 200

NW = 16
BPW = B // NW
LA = 96
LB = L - LA

VBLK = 8192

_mesh = plsc.VectorSubcoreMesh(
    core_axis_name="c", subcore_axis_name="s", num_cores=1)


def _tw_body(w_ref, t_ref, o_ref):
    o_ref[...] = jax.lax.dot_general(
        w_ref[...], t_ref[...], (((1,), (1,)), ((), ())),
        precision=jax.lax.Precision.DEFAULT)


_tw_kernel = pl.pallas_call(
    _tw_body,
    grid=((V + VBLK - 1) // VBLK,),
    in_specs=[
        pl.BlockSpec((8, D), lambda i: (0, 0)),
        pl.BlockSpec((VBLK, D), lambda i: (i, 0)),
    ],
    out_specs=pl.BlockSpec((8, VBLK), lambda i: (0, i)),
    out_shape=jax.ShapeDtypeStruct((8, V), jnp.float32),
)


@functools.partial(
    pl.kernel,
    out_type=jax.ShapeDtypeStruct((B * LABELS,), jnp.float32),
    mesh=_mesh,
    compiler_params=pltpu.CompilerParams(
        needs_layout_passes=False, use_tc_tiling_on_sc=False),
    scratch_types=[
        pltpu.VMEM((2, L), jnp.int32),
        pltpu.VMEM((2, L), jnp.float32),
        pltpu.VMEM((2, L), jnp.float32),
        pltpu.VMEM((BPW + 16,), jnp.float32),
        pltpu.VMEM((16,), jnp.float32),
        pltpu.VMEM((BPW * 16,), jnp.float32),
        pltpu.VMEM((BPW * LABELS,), jnp.float32),
        pltpu.SemaphoreType.DMA,
        pltpu.SemaphoreType.DMA,
        pltpu.SemaphoreType.DMA,
        pltpu.SemaphoreType.DMA,
    ],
)
def _embd_sc_kernel(x_hbm, xlen_hbm, tw0_hbm, tw1_hbm, b_hbm, out_hbm,
                    idx_v, g0, g1, xlen_v, b_v, tmp_v, out_v,
                    gsem0, gsem1, isem0, isem1):
    wid = lax.axis_index("s")
    base = wid * BPW

    pltpu.sync_copy(xlen_hbm.at[pl.ds(base, BPW)], xlen_v.at[pl.ds(0, BPW)])
    pltpu.sync_copy(b_hbm, b_v)

    gsem = (gsem0, gsem1)
    isem = (isem0, isem1)

    bvec = b_v[...]
    b0 = bvec[0]
    b1 = bvec[1]
    lane = lax.iota(jnp.int32, 16)
    tail_mask = lane >= 8

    def idx_copy(r, slot):
        r = jnp.minimum(r, BPW - 1)
        return pltpu.make_async_copy(
            x_hbm.at[pl.ds((base + r) * L, L)], idx_v.at[slot], isem[slot])

    def gathers(slot):
        yield pltpu.make_async_copy(
            tw0_hbm.at[idx_v.at[slot, pl.ds(0, LA)]],
            g0.at[slot, pl.ds(0, LA)], gsem[slot])
        yield pltpu.make_async_copy(
            tw0_hbm.at[idx_v.at[slot, pl.ds(LA, LB)]],
            g0.at[slot, pl.ds(LA, LB)], gsem[slot])
        yield pltpu.make_async_copy(
            tw1_hbm.at[idx_v.at[slot, pl.ds(0, LA)]],
            g1.at[slot, pl.ds(0, LA)], gsem[slot])
        yield pltpu.make_async_copy(
            tw1_hbm.at[idx_v.at[slot, pl.ds(LA, LB)]],
            g1.at[slot, pl.ds(LA, LB)], gsem[slot])

    def gather_start(slot):
        for cp in gathers(slot):
            cp.start()

    def gather_wait(slot):
        for cp in gathers(slot):
            cp.wait()

    def reduce_row(slot, r):
        z = jnp.zeros((16,), jnp.float32)

        def red(i, accs):
            a0, a1 = accs
            a0 = a0 + g0[slot, pl.ds(i * 16, 16)]
            a1 = a1 + g1[slot, pl.ds(i * 16, 16)]
            return (a0, a1)

        a0, a1 = lax.fori_loop(0, L // 16, red, (z, z), unroll=12)
        t0 = g0[slot, pl.ds(L - 16, 16)]
        t1 = g1[slot, pl.ds(L - 16, 16)]
        a0 = a0 + jnp.where(tail_mask, t0, 0.0)
        a1 = a1 + jnp.where(tail_mask, t1, 0.0)

        inv = (1.0 / xlen_v[pl.ds(r, 16)])[0]
        s0 = jnp.sum(a0) * inv + b0
        s1 = jnp.sum(a1) * inv + b1
        vres = jnp.where(lane == 0, jnp.full((16,), s0, jnp.float32),
                         jnp.full((16,), s1, jnp.float32))
        vres = 1.0 / (1.0 + jnp.exp(-vres))
        tmp_v[pl.ds(r * 16, 16)] = vres

    idx_copy(0, 0).start()
    idx_copy(1, 1).start()
    idx_copy(0, 0).wait()
    gather_start(0)
    idx_copy(1, 1).wait()
    gather_start(1)

    def pair_body(rp, _):
        r0 = 2 * rp
        gather_wait(0)
        idx_copy(r0 + 2, 0).start()
        reduce_row(0, r0)
        gather_wait(1)
        idx_copy(r0 + 3, 1).start()
        idx_copy(r0 + 2, 0).wait()
        gather_start(0)
        reduce_row(1, r0 + 1)
        idx_copy(r0 + 3, 1).wait()
        gather_start(1)
        return 0

    lax.fori_loop(0, BPW // 2, pair_body, 0)

    gather_wait(0)
    gather_wait(1)

    def pack_body(g, _):
        idx = ((lane >> 1) + g * 8) * 16 + (lane & 1)
        out_v[pl.ds(g * 16, 16)] = plsc.load_gather(tmp_v, [idx])
        return 0

    lax.fori_loop(0, (BPW * LABELS) // 16, pack_body, 0)

    pltpu.sync_copy(out_v, out_hbm.at[pl.ds(base * LABELS, BPW * LABELS)])


def kernel(x, xlength, embd_table, fc_W, fc_b):
    w_pad = jnp.zeros((8, D), jnp.float32).at[:LABELS].set(fc_W)
    tw = _tw_kernel(w_pad, embd_table)
    tw0 = tw[0]
    tw1 = tw[1]
    x_flat = x.reshape(B * L)
    xlen_flat = xlength.reshape(B)
    b_pad = jnp.zeros((16,), jnp.float32).at[:LABELS].set(fc_b)
    out_flat = _embd_sc_kernel(x_flat, xlen_flat, tw0, tw1, b_pad)
    return out_flat.reshape(B, LABELS)


# speedup vs baseline: 1.8786x; 1.1442x over previous
import functools

import jax
import jax.numpy as jnp
from jax import lax
from jax.experimental import pallas as pl
from jax.experimental.pallas import tpu as pltpu
from jax.experimental.pallas import tpu_sc as plsc

V = 1000000
D = 64
LABELS = 2
B = 4096
L = 200

NW = 32
BPW = B // NW
LA = 96
LB = L - LA

VBLK = 8192

_mesh = plsc.VectorSubcoreMesh(core_axis_name="c", subcore_axis_name="s")


def _tw_body(w_ref, t_ref, o_ref):
    o_ref[...] = jax.lax.dot_general(
        w_ref[...], t_ref[...], (((1,), (1,)), ((), ())),
        precision=jax.lax.Precision.DEFAULT)[:LABELS, :]


_tw_kernel = pl.pallas_call(
    _tw_body,
    grid=((V + VBLK - 1) // VBLK,),
    in_specs=[
        pl.BlockSpec((8, D), lambda i: (0, 0)),
        pl.BlockSpec((VBLK, D), lambda i: (i, 0)),
    ],
    out_specs=pl.BlockSpec((LABELS, VBLK), lambda i: (0, i)),
    out_shape=jax.ShapeDtypeStruct((LABELS, V), jnp.float32),
)


@functools.partial(
    pl.kernel,
    out_type=jax.ShapeDtypeStruct((B * LABELS,), jnp.float32),
    mesh=_mesh,
    compiler_params=pltpu.CompilerParams(
        needs_layout_passes=False, use_tc_tiling_on_sc=False),
    scratch_types=[
        pltpu.VMEM((2, L), jnp.int32),
        pltpu.VMEM((2, L), jnp.float32),
        pltpu.VMEM((2, L), jnp.float32),
        pltpu.VMEM((BPW + 16,), jnp.float32),
        pltpu.VMEM((16,), jnp.float32),
        pltpu.VMEM((BPW * 16,), jnp.float32),
        pltpu.VMEM((BPW * LABELS,), jnp.float32),
        pltpu.SemaphoreType.DMA,
        pltpu.SemaphoreType.DMA,
        pltpu.SemaphoreType.DMA,
        pltpu.SemaphoreType.DMA,
    ],
)
def _embd_sc_kernel(x_hbm, xlen_hbm, tw0_hbm, tw1_hbm, b_hbm, out_hbm,
                    idx_v, g0, g1, xlen_v, b_v, tmp_v, out_v,
                    gsem0, gsem1, isem0, isem1):
    wid = lax.axis_index("s") * 2 + lax.axis_index("c")
    base = wid * BPW

    pltpu.sync_copy(xlen_hbm.at[pl.ds(base, BPW)], xlen_v.at[pl.ds(0, BPW)])
    pltpu.sync_copy(b_hbm, b_v)

    gsem = (gsem0, gsem1)
    isem = (isem0, isem1)

    bvec = b_v[...]
    b0 = bvec[0]
    b1 = bvec[1]
    lane = lax.iota(jnp.int32, 16)
    tail_mask = lane >= 8

    def idx_copy(r, slot):
        r = jnp.minimum(r, BPW - 1)
        return pltpu.make_async_copy(
            x_hbm.at[pl.ds((base + r) * L, L)], idx_v.at[slot], isem[slot])

    def gathers(slot):
        yield pltpu.make_async_copy(
            tw0_hbm.at[idx_v.at[slot, pl.ds(0, LA)]],
            g0.at[slot, pl.ds(0, LA)], gsem[slot])
        yield pltpu.make_async_copy(
            tw0_hbm.at[idx_v.at[slot, pl.ds(LA, LB)]],
            g0.at[slot, pl.ds(LA, LB)], gsem[slot])
        yield pltpu.make_async_copy(
            tw1_hbm.at[idx_v.at[slot, pl.ds(0, LA)]],
            g1.at[slot, pl.ds(0, LA)], gsem[slot])
        yield pltpu.make_async_copy(
            tw1_hbm.at[idx_v.at[slot, pl.ds(LA, LB)]],
            g1.at[slot, pl.ds(LA, LB)], gsem[slot])

    def gather_start(slot):
        for cp in gathers(slot):
            cp.start()

    def gather_wait(slot):
        for cp in gathers(slot):
            cp.wait()

    def reduce_row(slot, r):
        z = jnp.zeros((16,), jnp.float32)

        def red(i, accs):
            a0, a1 = accs
            a0 = a0 + g0[slot, pl.ds(i * 16, 16)]
            a1 = a1 + g1[slot, pl.ds(i * 16, 16)]
            return (a0, a1)

        a0, a1 = lax.fori_loop(0, L // 16, red, (z, z), unroll=12)
        t0 = g0[slot, pl.ds(L - 16, 16)]
        t1 = g1[slot, pl.ds(L - 16, 16)]
        a0 = a0 + jnp.where(tail_mask, t0, 0.0)
        a1 = a1 + jnp.where(tail_mask, t1, 0.0)

        inv = (1.0 / xlen_v[pl.ds(r, 16)])[0]
        s0 = jnp.sum(a0) * inv + b0
        s1 = jnp.sum(a1) * inv + b1
        vres = jnp.where(lane == 0, jnp.full((16,), s0, jnp.float32),
                         jnp.full((16,), s1, jnp.float32))
        vres = 1.0 / (1.0 + jnp.exp(-vres))
        tmp_v[pl.ds(r * 16, 16)] = vres

    idx_copy(0, 0).start()
    idx_copy(1, 1).start()
    idx_copy(0, 0).wait()
    gather_start(0)
    idx_copy(1, 1).wait()
    gather_start(1)

    def pair_body(rp, _):
        r0 = 2 * rp
        gather_wait(0)
        idx_copy(r0 + 2, 0).start()
        reduce_row(0, r0)
        gather_wait(1)
        idx_copy(r0 + 3, 1).start()
        idx_copy(r0 + 2, 0).wait()
        gather_start(0)
        reduce_row(1, r0 + 1)
        idx_copy(r0 + 3, 1).wait()
        gather_start(1)
        return 0

    lax.fori_loop(0, BPW // 2, pair_body, 0)

    gather_wait(0)
    gather_wait(1)

    def pack_body(g, _):
        idx = ((lane >> 1) + g * 8) * 16 + (lane & 1)
        out_v[pl.ds(g * 16, 16)] = plsc.load_gather(tmp_v, [idx])
        return 0

    lax.fori_loop(0, (BPW * LABELS) // 16, pack_body, 0)

    pltpu.sync_copy(out_v, out_hbm.at[pl.ds(base * LABELS, BPW * LABELS)])


def kernel(x, xlength, embd_table, fc_W, fc_b):
    w_pad = jnp.zeros((8, D), jnp.float32).at[:LABELS].set(fc_W)
    tw = _tw_kernel(w_pad, embd_table)
    tw0 = tw[0]
    tw1 = tw[1]
    x_flat = x.reshape(B * L)
    xlen_flat = xlength.reshape(B)
    b_pad = jnp.zeros((16,), jnp.float32).at[:LABELS].set(fc_b)
    out_flat = _embd_sc_kernel(x_flat, xlen_flat, tw0, tw1, b_pad)
    return out_flat.reshape(B, LABELS)
